# baseline (speedup 1.0000x reference)
import functools

import jax
import jax.numpy as jnp
from jax import lax
from jax.experimental import pallas as pl
from jax.experimental.pallas import tpu as pltpu
from jax.experimental.pallas import tpu_sc as plsc

_C = 10000
_CP = 10240
_F = 128
_N = 320000
_NC = 2
_NS = 16
_NW = _NC * _NS
_ROWS_W = _N // _NW
_CH = 80
_NCH = _ROWS_W // _CH
_CPT = _CP // _NS

_HR = 640
_HNB = _N // _HR
_HW = 64


def _sc_mesh():
    return plsc.VectorSubcoreMesh(core_axis_name="c", subcore_axis_name="s")


@functools.partial(
    pl.kernel,
    out_type=jax.ShapeDtypeStruct((_NC, _CP, _F), jnp.float32),
    mesh=_sc_mesh(),
    scratch_types=[
        pltpu.VMEM_SHARED((_CP, _F), jnp.float32),
        pltpu.VMEM((_CH, _F), jnp.float32),
        pltpu.VMEM((_CH, _F), jnp.float32),
        pltpu.VMEM((_CH,), jnp.int32),
        pltpu.VMEM((_CH,), jnp.int32),
        pltpu.SemaphoreType.DMA,
        pltpu.SemaphoreType.DMA,
    ],
)
def _sc_segsum(x_hbm, lab_hbm, zsum_hbm, psum_hbm, acc,
               rows0, rows1, idx0, idx1, semg0, semg1):
    cid = lax.axis_index("c")
    sid = lax.axis_index("s")
    wid = cid * _NS + sid

    pltpu.sync_copy(zsum_hbm, acc.at[pl.ds(sid * _CPT, _CPT)])
    plsc.subcore_barrier()

    base = wid * _ROWS_W
    bufs = ((rows0, idx0, semg0), (rows1, idx1, semg1))

    def _start(j, rows, idx, semg):
        off = base + j * _CH
        pltpu.async_copy(x_hbm.at[pl.ds(off, _CH)], rows, semg)
        pltpu.async_copy(lab_hbm.at[pl.ds(off, _CH)], idx, semg)

    def _wait(rows, idx, semg):
        pltpu.make_async_copy(x_hbm.at[pl.ds(0, _CH)], rows, semg).wait()
        pltpu.make_async_copy(lab_hbm.at[pl.ds(0, _CH)], idx, semg).wait()

    _start(0, *bufs[0])

    def _pair(jj, carry):
        for k in (0, 1):
            j = jj * 2 + k
            rows, idx, semg = bufs[k]
            orows, oidx, osemg = bufs[1 - k]

            _wait(rows, idx, semg)

            @pl.when(j + 1 < _NCH)
            def _prefetch():
                _start(j + 1, orows, oidx, osemg)

            pltpu.sync_copy(rows, acc.at[idx], add=True)
        return carry

    lax.fori_loop(0, _NCH // 2, _pair, 0)

    rows, idx, semg = bufs[0]
    _wait(rows, idx, semg)
    pltpu.sync_copy(rows, acc.at[idx], add=True)
    plsc.subcore_barrier()

    r0 = sid * _CPT
    pltpu.sync_copy(acc.at[pl.ds(r0, _CPT)], psum_hbm.at[cid, pl.ds(r0, _CPT)])


def _hist_body(lab_ref, cnt_ref, cac):
    cac[...] = jnp.zeros_like(cac)

    def _blk(j, carry):
        labs_row = lab_ref[j]
        lo = labs_row[0, 0]
        hi = labs_row[0, _HR - 1]
        base0 = (lo // 8) * 8
        nwin = (hi - base0) // _HW + 1

        def _win(w, c2):
            cbase = base0 + w * _HW
            rowi = lax.broadcasted_iota(jnp.int32, (_HW, _HR), 0) + cbase
            onehot_t = (rowi == labs_row).astype(jnp.float32)
            cac[pl.ds(cbase, _HW), :] += jnp.sum(
                onehot_t, axis=1, keepdims=True)
            return c2

        lax.fori_loop(0, nwin, _win, 0)
        return carry

    lax.fori_loop(0, _HNB, _blk, 0)
    cnt_ref[...] = cac[: _C, :]


def _hist(labels3):
    return pl.pallas_call(
        _hist_body,
        out_shape=jax.ShapeDtypeStruct((_C, 1), jnp.float32),
        scratch_shapes=[pltpu.VMEM((_CP + _HW, 1), jnp.float32)],
    )(labels3)


_DB = 1000
_DNB = _C // _DB


def _divide_body(psum_ref, cntf_ref, mean_ref, cnt_ref):
    cnt = cntf_ref[...]
    sums = psum_ref[0] + psum_ref[1]
    mean_ref[...] = sums / jnp.maximum(cnt, 1.0)
    cnt_ref[...] = cnt.astype(jnp.int32)


def _divide(psum, cntf):
    return pl.pallas_call(
        _divide_body,
        grid=(_DNB,),
        in_specs=[
            pl.BlockSpec((_NC, _DB, _F), lambda b: (0, b, 0)),
            pl.BlockSpec((_DB, 1), lambda b: (b, 0)),
        ],
        out_specs=[
            pl.BlockSpec((_DB, _F), lambda b: (b, 0)),
            pl.BlockSpec((_DB, 1), lambda b: (b, 0)),
        ],
        out_shape=[
            jax.ShapeDtypeStruct((_C, _F), jnp.float32),
            jax.ShapeDtypeStruct((_C, 1), jnp.int32),
        ],
    )(psum, cntf)


def kernel(inputs, labels):
    labels_i = labels.astype(jnp.int32)
    zsum = jnp.zeros((_CPT, _F), jnp.float32)
    psum = _sc_segsum(inputs, labels_i, zsum)
    cntf = _hist(labels_i.reshape(_HNB, 1, _HR))
    mean, cnt = _divide(psum, cntf)
    return (inputs, mean, cnt.reshape(_C))


# speedup vs baseline: 1.1297x; 1.1297x over previous
import functools

import jax
import jax.numpy as jnp
from jax import lax
from jax.experimental import pallas as pl
from jax.experimental.pallas import tpu as pltpu
from jax.experimental.pallas import tpu_sc as plsc

_C = 10000
_CP = 10240
_F = 128
_N = 320000
_NC = 2
_NS = 16
_NW = _NC * _NS
_ROWS_W = _N // _NW
_CH = 80
_NCH = _ROWS_W // _CH
_CPT = _CP // _NS

_HR = 640
_HNB = _N // _HR
_HW = 64


def _sc_mesh():
    return plsc.VectorSubcoreMesh(core_axis_name="c", subcore_axis_name="s")


@functools.partial(
    pl.kernel,
    out_type=jax.ShapeDtypeStruct((_NC, _CP, _F), jnp.float32),
    mesh=_sc_mesh(),
    scratch_types=[
        pltpu.VMEM_SHARED((_CP, _F), jnp.float32),
        pltpu.VMEM((_CH, _F), jnp.float32),
        pltpu.VMEM((_CH, _F), jnp.float32),
        pltpu.VMEM((_CH,), jnp.int32),
        pltpu.VMEM((_CH,), jnp.int32),
        pltpu.SemaphoreType.DMA,
        pltpu.SemaphoreType.DMA,
    ],
)
def _sc_segsum(x_hbm, lab_hbm, zsum_hbm, psum_hbm, acc,
               rows0, rows1, idx0, idx1, semg0, semg1):
    cid = lax.axis_index("c")
    sid = lax.axis_index("s")
    wid = cid * _NS + sid

    pltpu.sync_copy(zsum_hbm, acc.at[pl.ds(sid * _CPT, _CPT)])
    plsc.subcore_barrier()

    base = wid * _ROWS_W
    bufs = ((rows0, idx0, semg0), (rows1, idx1, semg1))

    def _start(j, rows, idx, semg):
        off = base + j * _CH
        pltpu.async_copy(x_hbm.at[pl.ds(off, _CH)], rows, semg)
        pltpu.async_copy(lab_hbm.at[pl.ds(off, _CH)], idx, semg)

    def _wait(rows, idx, semg):
        pltpu.make_async_copy(x_hbm.at[pl.ds(0, _CH)], rows, semg).wait()
        pltpu.make_async_copy(lab_hbm.at[pl.ds(0, _CH)], idx, semg).wait()

    _start(0, *bufs[0])

    def _pair(jj, carry):
        for k in (0, 1):
            j = jj * 2 + k
            rows, idx, semg = bufs[k]
            orows, oidx, osemg = bufs[1 - k]

            _wait(rows, idx, semg)

            @pl.when(j + 1 < _NCH)
            def _prefetch():
                _start(j + 1, orows, oidx, osemg)

            pltpu.sync_copy(rows, acc.at[idx], add=True)
        return carry

    lax.fori_loop(0, _NCH // 2, _pair, 0)

    rows, idx, semg = bufs[0]
    _wait(rows, idx, semg)
    pltpu.sync_copy(rows, acc.at[idx], add=True)
    plsc.subcore_barrier()

    r0 = sid * _CPT
    pltpu.sync_copy(acc.at[pl.ds(r0, _CPT)], psum_hbm.at[cid, pl.ds(r0, _CPT)])


_XB = 16000
_XNB = _N // _XB
_HPB = _HNB // _XNB


def _histcopy_body(x_ref, lab_ref, xout_ref, cnt_ref, cac):
    b = pl.program_id(0)

    @pl.when(b == 0)
    def _init():
        cac[...] = jnp.zeros_like(cac)

    xout_ref[...] = x_ref[...]

    def _blk(jj, carry):
        labs_row = lab_ref[b * _HPB + jj]
        lo = labs_row[0, 0]
        hi = labs_row[0, _HR - 1]
        base0 = (lo // 8) * 8
        nwin = (hi - base0) // _HW + 1

        def _win(w, c2):
            cbase = base0 + w * _HW
            rowi = lax.broadcasted_iota(jnp.int32, (_HW, _HR), 0) + cbase
            onehot_t = (rowi == labs_row).astype(jnp.float32)
            cac[pl.ds(cbase, _HW), :] += jnp.sum(
                onehot_t, axis=1, keepdims=True)
            return c2

        lax.fori_loop(0, nwin, _win, 0)
        return carry

    lax.fori_loop(0, _HPB, _blk, 0)

    @pl.when(b == _XNB - 1)
    def _fin():
        cnt_ref[...] = cac[: _C, :]


def _histcopy(inputs, labels3):
    return pl.pallas_call(
        _histcopy_body,
        grid=(_XNB,),
        in_specs=[
            pl.BlockSpec((_XB, _F), lambda b: (b, 0)),
            pl.BlockSpec((_HNB, 1, _HR), lambda b: (0, 0, 0)),
        ],
        out_specs=[
            pl.BlockSpec((_XB, _F), lambda b: (b, 0)),
            pl.BlockSpec((_C, 1), lambda b: (0, 0)),
        ],
        out_shape=[
            jax.ShapeDtypeStruct((_N, _F), jnp.float32),
            jax.ShapeDtypeStruct((_C, 1), jnp.float32),
        ],
        scratch_shapes=[pltpu.VMEM((_CP + _HW, 1), jnp.float32)],
    )(inputs, labels3)


_DB = 1000
_DNB = _C // _DB


def _divide_body(psum_ref, cntf_ref, mean_ref, cnt_ref):
    cnt = cntf_ref[...]
    sums = psum_ref[0] + psum_ref[1]
    mean_ref[...] = sums / jnp.maximum(cnt, 1.0)
    cnt_ref[...] = cnt.astype(jnp.int32)


def _divide(psum, cntf):
    return pl.pallas_call(
        _divide_body,
        grid=(_DNB,),
        in_specs=[
            pl.BlockSpec((_NC, _DB, _F), lambda b: (0, b, 0)),
            pl.BlockSpec((_DB, 1), lambda b: (b, 0)),
        ],
        out_specs=[
            pl.BlockSpec((_DB, _F), lambda b: (b, 0)),
            pl.BlockSpec((_DB, 1), lambda b: (b, 0)),
        ],
        out_shape=[
            jax.ShapeDtypeStruct((_C, _F), jnp.float32),
            jax.ShapeDtypeStruct((_C, 1), jnp.int32),
        ],
    )(psum, cntf)


def kernel(inputs, labels):
    labels_i = labels.astype(jnp.int32)
    zsum = jnp.zeros((_CPT, _F), jnp.float32)
    psum = _sc_segsum(inputs, labels_i, zsum)
    xout, cntf = _histcopy(inputs, labels_i.reshape(_HNB, 1, _HR))
    mean, cnt = _divide(psum, cntf)
    return (xout, mean, cnt.reshape(_C))
